# baseline (speedup 1.0000x reference)
import functools

import jax
import jax.numpy as jnp
from jax import lax
from jax.experimental import pallas as pl
from jax.experimental.pallas import tpu as pltpu
from jax.experimental.pallas import tpu_sc as plsc

EMBED_DIM = 128
NUM_CORES = 2
NUM_SUBCORES = 16
NW = NUM_CORES * NUM_SUBCORES
CHUNK = 128
NT = 4
NS = 2
PERIOD = 4


@functools.lru_cache(maxsize=None)
def _make_gather(total, dim):
    per_w = total // NW
    n_chunks = per_w // CHUNK
    assert n_chunks >= 2 * PERIOD
    mesh = plsc.VectorSubcoreMesh(core_axis_name="c", subcore_axis_name="s")

    @functools.partial(
        pl.kernel,
        out_type=jax.ShapeDtypeStruct((total, dim), jnp.float32),
        mesh=mesh,
        scratch_types=[
            pltpu.VMEM((n_chunks, CHUNK), jnp.int32),
            pltpu.VMEM((NT, CHUNK, dim), jnp.float32),
            pltpu.VMEM_SHARED((NUM_SUBCORES, NS, CHUNK, dim), jnp.float32),
            pltpu.SemaphoreType.DMA((NT,)),
            pltpu.SemaphoreType.DMA((NS,)),
            pltpu.SemaphoreType.DMA((NS,)),
        ],
    )
    def gather_kernel(idx_hbm, table_hbm, out_hbm, idx_v, rows_v, sp_v,
                      sem_g, sem_d, sem_s):
        sid = lax.axis_index("s")
        wid = sid * NUM_CORES + lax.axis_index("c")
        base = wid * per_w

        pltpu.sync_copy(idx_hbm.at[wid], idx_v)

        def gather(c, a):
            pltpu.async_copy(table_hbm.at[idx_v.at[c]], rows_v.at[a],
                             sem_g.at[a])

        def wait_gather(a):
            pltpu.make_async_copy(table_hbm.at[idx_v.at[0]], rows_v.at[a],
                                  sem_g.at[a]).wait()

        def dma(a, p):
            pltpu.async_copy(rows_v.at[a], sp_v.at[sid, p], sem_d.at[p])

        def wait_dma(p):
            pltpu.make_async_copy(rows_v.at[0], sp_v.at[sid, p],
                                  sem_d.at[p]).wait()

        def store(c, p):
            pltpu.async_copy(sp_v.at[sid, p],
                             out_hbm.at[pl.ds(base + c * CHUNK, CHUNK)],
                             sem_s.at[p])

        def wait_store(p):
            pltpu.make_async_copy(sp_v.at[sid, p], out_hbm.at[pl.ds(base, CHUNK)],
                                  sem_s.at[p]).wait()

        def step(t, a, p, do_wait_store=True, do_store=True, do_gather=True):
            wait_gather(a)
            if do_wait_store:
                wait_store(p)
            dma(a, p)
            if do_store:
                wait_dma((p + NS - 1) % NS)
                store(t - 1, (p + NS - 1) % NS)
            if do_gather:
                gather(t + NT - 1, (a + NT - 1) % NT)

        for c in range(NT - 1):
            gather(c, c)
        for t in range(NT):
            step(t, t % NT, t % NS, do_wait_store=(t >= NS), do_store=(t >= 1))

        n_steady = n_chunks - 2 * NT + 1
        n_blocks = n_steady // PERIOD

        def body(k, carry):
            t0 = NT + PERIOD * k
            for b in range(PERIOD):
                step(t0 + b, (NT + b) % NT, (NT + b) % NS)
            return carry

        lax.fori_loop(0, n_blocks, body, 0)

        for t in range(NT + PERIOD * n_blocks, n_chunks - NT + 1):
            step(t, t % NT, t % NS)

        for t in range(n_chunks - NT + 1, n_chunks):
            step(t, t % NT, t % NS, do_gather=False)
        p = (n_chunks - 1) % NS
        wait_dma(p)
        store(n_chunks - 1, p)
        for d in range(1, NS):
            wait_store((p + d) % NS)
        wait_store(p)

    return gather_kernel


def kernel(indices, table):
    batch, tokens = indices.shape
    total = batch * tokens
    per_w = total // NW
    idx3 = indices.reshape(NW, per_w // CHUNK, CHUNK).astype(jnp.int32)
    out = _make_gather(total, table.shape[1])(idx3, table)
    return out.reshape(batch, tokens, table.shape[1])
